# baseline (speedup 1.0000x reference)
import functools

import jax
import jax.numpy as jnp
from jax import lax
from jax.experimental import pallas as pl
from jax.experimental.pallas import tpu as pltpu
from jax.experimental.pallas import tpu_sc as plsc

N = 10000
E = 320000
NUM_ELEM = 10
C = 16
NG = 16
R_MAX = 5.0
NUM_BESSEL = 8
AVG_NEIGH = 32.0
SH_DIM = 9
AROW = SH_DIM * C

NC = 2
NS = 16
NW = NC * NS
EPW = E // NW
NPT = N // NS

@functools.cache
def _sc_mesh():
    return plsc.VectorSubcoreMesh(core_axis_name="c", subcore_axis_name="s",
                                  num_cores=NC, num_subcores=NS)


_SC_PARAMS = pltpu.CompilerParams(needs_layout_passes=False,
                                  use_tc_tiling_on_sc=False)


def _silu(x):
    return x / (1.0 + jnp.exp(-x))


def _geom_body(estart, ne, pos_hbm, ei_hbm, dvec_hbm, pos_v, sidx_v,
               ridx_v, dbuf_v):
    epw = ne // NW
    wid = lax.axis_index("s") * NC + lax.axis_index("c")
    base = wid * epw
    gbase = estart + base
    pltpu.sync_copy(pos_hbm, pos_v)
    pltpu.sync_copy(ei_hbm.at[0, pl.ds(gbase, epw)], sidx_v)
    pltpu.sync_copy(ei_hbm.at[1, pl.ds(gbase, epw)], ridx_v)

    def grp(g, _):
        off = g * 16
        si = sidx_v[pl.ds(off, 16)] * 4
        ri = ridx_v[pl.ds(off, 16)] * 4
        for c in range(3):
            ps = plsc.load_gather(pos_v, [si + c])
            pr = plsc.load_gather(pos_v, [ri + c])
            dbuf_v[c, pl.ds(off, 16)] = pr - ps
        return 0

    lax.fori_loop(0, epw // 16, grp, 0)
    pltpu.sync_copy(dbuf_v, dvec_hbm.at[:, pl.ds(base, epw)])


def _geom_call(pos4flat, edge_index, estart, ne):
    epw = ne // NW
    return pl.kernel(
        functools.partial(_geom_body, estart, ne),
        out_type=jax.ShapeDtypeStruct((3, ne), jnp.float32),
        mesh=_sc_mesh(),
        compiler_params=_SC_PARAMS,
        scratch_types=[
            pltpu.VMEM((N * 4,), jnp.float32),
            pltpu.VMEM((epw,), jnp.int32),
            pltpu.VMEM((epw,), jnp.int32),
            pltpu.VMEM((3, epw), jnp.float32),
        ],
    )(pos4flat, edge_index)


def _edge_body(dvec_ref, shifts_ref, w10, w20, w30, out_ref):
    x = dvec_ref[0:1, :] + shifts_ref[0:1, :]
    y = dvec_ref[1:2, :] + shifts_ref[1:2, :]
    z = dvec_ref[2:3, :] + shifts_ref[2:3, :]
    r2 = x * x + y * y + z * z + 1e-12
    r = jnp.sqrt(r2)
    inv_r = 1.0 / r
    ux = x * inv_r
    uy = y * inv_r
    uz = z * inv_r
    s3 = 3.0 ** 0.5
    s5 = 5.0 ** 0.5
    s15 = 15.0 ** 0.5
    sh = jnp.concatenate([
        s3 * ux, s3 * uy, s3 * uz,
        s15 * ux * uy, s15 * uy * uz, 0.5 * s5 * (3.0 * uz * uz - 1.0),
        s15 * ux * uz, 0.5 * s15 * (ux * ux - uy * uy)], axis=0)
    n = (lax.broadcasted_iota(jnp.int32, (NUM_BESSEL, 1), 0) + 1
         ).astype(jnp.float32)
    pref = (2.0 / R_MAX) ** 0.5
    rb = pref * jnp.sin(n * (jnp.pi / R_MAX) * r) * inv_r
    xc = r * (1.0 / R_MAX)
    x2 = xc * xc
    x4 = x2 * x2
    x5 = x4 * xc
    x6 = x5 * xc
    x7 = x6 * xc
    env = 1.0 - 21.0 * x5 + 35.0 * x6 - 15.0 * x7
    env = jnp.where(xc < 1.0, env, 0.0)
    rb = rb * env
    g = jnp.concatenate([rb, sh], axis=0)
    gt = jnp.transpose(g, (1, 0))
    rb = gt[:, 0:NUM_BESSEL]
    sh = gt[:, NUM_BESSEL:16]
    rr = _silu(jnp.dot(rb, w10[...], preferred_element_type=jnp.float32))
    rr = _silu(jnp.dot(rr, w20[...], preferred_element_type=jnp.float32))
    rr = jnp.dot(rr, w30[...], preferred_element_type=jnp.float32)
    out_ref[:, 0:48] = rr[:, 0:48]
    out_ref[:, 48:56] = sh
    out_ref[:, 56:64] = jnp.zeros_like(out_ref[:, 56:64])
    out_ref[:, 64:112] = rr[:, 48:96]
    out_ref[:, 112:120] = sh
    out_ref[:, 120:128] = jnp.zeros_like(out_ref[:, 120:128])


def _wspec(a):
    return pl.BlockSpec(a.shape, lambda i: (0,) * a.ndim)


def _edge_call(dvec, shifts4, w10, w20, w30, lo, ne):
    eb = 1280
    ofs = lo // eb
    return pl.pallas_call(
        _edge_body,
        grid=ne // eb,
        in_specs=[
            pl.BlockSpec((3, eb), lambda i: (0, i)),
            pl.BlockSpec((3, eb), lambda i: (0, i + ofs)),
            _wspec(w10), _wspec(w20), _wspec(w30),
        ],
        out_specs=pl.BlockSpec((eb, 128), lambda i: (i, 0)),
        out_shape=jax.ShapeDtypeStruct((ne, 128), jnp.float32),
    )(dvec, shifts4, w10, w20, w30)


CHUNK = 80


def _msg_body(roff, estart, epw, chunk, chained, edata_hbm, htab_hbm,
              ei3_hbm, init_hbm, apart_hbm, ed0, ed1, hs0, hs1, eib0,
              eib1, msg0, msg1, acc_sh, se0, se1, si0, si1, sg0, sg1,
              sc0, sc1):
    nch = epw // chunk
    cid = lax.axis_index("c")
    sid = lax.axis_index("s")
    wid = sid * NC + cid
    base = wid * epw
    gc0 = (estart + base) // chunk
    ed = (ed0, ed1)
    hs = (hs0, hs1)
    eib = (eib0, eib1)
    msg = (msg0, msg1)
    se = (se0, se1)
    si = (si0, si1)
    sg = (sg0, sg1)
    sc = (sc0, sc1)

    for t in range(NS):
        lo = 624 * t
        sz = 624 if t < NS - 1 else N - 624 * (NS - 1)

        @pl.when(sid == t)
        def _(lo=lo, sz=sz):
            if chained:
                pltpu.sync_copy(init_hbm.at[cid, pl.ds(lo, sz)],
                                acc_sh.at[pl.ds(lo, sz)])
            else:
                pltpu.sync_copy(init_hbm.at[pl.ds(lo, sz)],
                                acc_sh.at[pl.ds(lo, sz)])

    plsc.subcore_barrier()

    def lin_issue(j, b):
        cb = base + j * chunk
        pltpu.async_copy(edata_hbm.at[pl.ds(cb, chunk), pl.ds(roff, 64)],
                         ed[b], se[b])
        pltpu.async_copy(ei3_hbm.at[:, gc0 + j, :], eib[b], si[b])

    def wait_si(b):
        pltpu.make_async_copy(ei3_hbm.at[:, gc0, :], eib[b],
                              si[b]).wait()

    def gather_issue(b):
        pltpu.async_copy(htab_hbm.at[eib[b].at[0]], hs[b], sg[b])

    def process(b):
        pltpu.make_async_copy(
            edata_hbm.at[pl.ds(base, chunk), pl.ds(roff, 64)], ed[b],
            se[b]).wait()
        pltpu.make_async_copy(htab_hbm.at[eib[b].at[0]], hs[b],
                              sg[b]).wait()
        ed_v = ed[b]
        hs_v = hs[b]
        msg_v = msg[b]

        def edge(i, _):
            hv = hs_v[i, :]
            t0 = ed_v[i, pl.ds(0, 16)] * hv
            t1 = ed_v[i, pl.ds(16, 16)] * hv
            t2 = ed_v[i, pl.ds(32, 16)] * hv
            shv = ed_v[i, pl.ds(48, 16)]
            msg_v[i, pl.ds(0, 16)] = t0
            msg_v[i, pl.ds(16, 16)] = t1 * shv[0]
            msg_v[i, pl.ds(32, 16)] = t1 * shv[1]
            msg_v[i, pl.ds(48, 16)] = t1 * shv[2]
            msg_v[i, pl.ds(64, 16)] = t2 * shv[3]
            msg_v[i, pl.ds(80, 16)] = t2 * shv[4]
            msg_v[i, pl.ds(96, 16)] = t2 * shv[5]
            msg_v[i, pl.ds(112, 16)] = t2 * shv[6]
            msg_v[i, pl.ds(128, 16)] = t2 * shv[7]
            return 0

        lax.fori_loop(0, chunk, edge, 0, unroll=2)
        pltpu.async_copy(msg_v, acc_sh.at[eib[b].at[1]], sc[b], add=True)

    def wait_sc(b):
        pltpu.make_async_copy(msg[b], acc_sh.at[eib[b].at[1]],
                              sc[b]).wait()

    lin_issue(0, 0)
    wait_si(0)
    gather_issue(0)
    lin_issue(1, 1)

    def pair(i, _):
        j = 2 * i
        wait_si(1)
        gather_issue(1)
        process(0)
        process(1)

        @pl.when(j + 2 < nch)
        def _():
            wait_sc(0)
            lin_issue(j + 2, 0)
            wait_si(0)
            gather_issue(0)

        @pl.when(j + 3 < nch)
        def _():
            wait_sc(1)
            lin_issue(j + 3, 1)

        return 0

    lax.fori_loop(0, nch // 2, pair, 0)
    if nch % 2 == 1:
        process(0)
    wait_sc(0)
    wait_sc(1)
    plsc.subcore_barrier()
    for t in range(NS):
        lo = 624 * t
        sz = 624 if t < NS - 1 else N - 624 * (NS - 1)

        @pl.when(sid == t)
        def _(lo=lo, sz=sz):
            pltpu.sync_copy(acc_sh.at[pl.ds(lo, sz)],
                            apart_hbm.at[cid, pl.ds(lo, sz)])


def _msg_call(roff, estart, epw, chunk, edata, htab, edge_index, init,
              chained=False):
    body = functools.partial(_msg_body, roff, estart, epw, chunk, chained)
    return pl.kernel(
        body,
        out_type=jax.ShapeDtypeStruct((NC, N, AROW), jnp.float32),
        mesh=_sc_mesh(),
        compiler_params=_SC_PARAMS,
        scratch_types=[
            pltpu.VMEM((chunk, 64), jnp.float32),
            pltpu.VMEM((chunk, 64), jnp.float32),
            pltpu.VMEM((chunk, 16), jnp.float32),
            pltpu.VMEM((chunk, 16), jnp.float32),
            pltpu.VMEM((2, chunk), jnp.int32),
            pltpu.VMEM((2, chunk), jnp.int32),
            pltpu.VMEM((chunk, AROW), jnp.float32),
            pltpu.VMEM((chunk, AROW), jnp.float32),
            pltpu.VMEM_SHARED((N, AROW), jnp.float32),
        ] + [pltpu.SemaphoreType.DMA] * 8,
    )(edata, htab, edge_index, init)


def _embed_body(na_ref, we_ref, ae_ref, h0_ref, e0_ref):
    na = na_ref[...]
    h0_ref[...] = jnp.dot(na, we_ref[...], preferred_element_type=jnp.float32)
    e0_ref[...] = jnp.dot(na, ae_ref[...], preferred_element_type=jnp.float32)


def _embed_call(node_attrs, w_embed, ae_col):
    nb = 1000
    return pl.pallas_call(
        _embed_body,
        grid=N // nb,
        in_specs=[
            pl.BlockSpec((nb, NUM_ELEM), lambda i: (i, 0)),
            _wspec(w_embed),
            _wspec(ae_col),
        ],
        out_specs=[
            pl.BlockSpec((nb, C), lambda i: (i, 0)),
            pl.BlockSpec((nb, 1), lambda i: (i, 0)),
        ],
        out_shape=[
            jax.ShapeDtypeStruct((N, C), jnp.float32),
            jax.ShapeDtypeStruct((N, 1), jnp.float32),
        ],
    )(node_attrs, w_embed, ae_col)


def _node_body(apart_ref, wbig_ref, mavg_ref, prod_ref,
               hold_ref, rw1_ref, rw2_ref, hnew_ref, e_ref):
    a = (apart_ref[0] + apart_ref[1]) * (1.0 / AVG_NEIGH)
    amix = jnp.dot(a, wbig_ref[...], preferred_element_type=jnp.float32)
    inv = jnp.dot(amix * amix, mavg_ref[...],
                  preferred_element_type=jnp.float32)
    s = (amix[:, 0:16] + jnp.dot(inv, prod_ref[...],
                                 preferred_element_type=jnp.float32)
         + hold_ref[...])
    hnew_ref[...] = s
    e_ref[...] = jnp.dot(s, rw1_ref[...], preferred_element_type=jnp.float32)


def _node_call(apart, wbig, mavg, prod, hold, rw1, rw2):
    nb = 1000
    return pl.pallas_call(
        _node_body,
        grid=N // nb,
        in_specs=[
            pl.BlockSpec((NC, nb, AROW), lambda i: (0, i, 0)),
            _wspec(wbig), _wspec(mavg), _wspec(prod),
            pl.BlockSpec((nb, C), lambda i: (i, 0)),
            _wspec(rw1), _wspec(rw2),
        ],
        out_specs=[
            pl.BlockSpec((nb, C), lambda i: (i, 0)),
            pl.BlockSpec((nb, 1), lambda i: (i, 0)),
        ],
        out_shape=[
            jax.ShapeDtypeStruct((N, C), jnp.float32),
            jax.ShapeDtypeStruct((N, 1), jnp.float32),
        ],
    )(apart, wbig, mavg, prod, hold, rw1, rw2)


def _node2_body(apart_ref, wbig_ref, mavg_ref, prod_ref,
                hold_ref, rw1_ref, rw2_ref, e0_ref, e1_ref, batch_ref,
                ne_ref, tot_ref):
    a = (apart_ref[0] + apart_ref[1]) * (1.0 / AVG_NEIGH)
    amix = jnp.dot(a, wbig_ref[...], preferred_element_type=jnp.float32)
    inv = jnp.dot(amix * amix, mavg_ref[...],
                  preferred_element_type=jnp.float32)
    s = (amix[:, 0:16] + jnp.dot(inv, prod_ref[...],
                                 preferred_element_type=jnp.float32)
         + hold_ref[...])
    t = _silu(jnp.dot(s, rw1_ref[...], preferred_element_type=jnp.float32))
    e2 = jnp.dot(t, rw2_ref[...], preferred_element_type=jnp.float32)
    ne = e0_ref[...] + e1_ref[...] + e2
    ne_ref[...] = ne
    gi = lax.broadcasted_iota(jnp.int32, (1, NG), 1)
    oh = (batch_ref[...] == gi).astype(jnp.float32)
    part = jnp.sum(oh * ne, axis=0, keepdims=True)

    @pl.when(pl.program_id(0) == 0)
    def _():
        tot_ref[...] = jnp.zeros_like(tot_ref)

    tot_ref[...] += part


def _node2_call(apart, wbig, mavg, prod, hold, rw1, rw2, e0, e1,
                batch2d):
    nb = 1000
    return pl.pallas_call(
        _node2_body,
        grid=N // nb,
        in_specs=[
            pl.BlockSpec((NC, nb, AROW), lambda i: (0, i, 0)),
            _wspec(wbig), _wspec(mavg), _wspec(prod),
            pl.BlockSpec((nb, C), lambda i: (i, 0)),
            _wspec(rw1), _wspec(rw2),
            pl.BlockSpec((nb, 1), lambda i: (i, 0)),
            pl.BlockSpec((nb, 1), lambda i: (i, 0)),
            pl.BlockSpec((nb, 1), lambda i: (i, 0)),
        ],
        out_specs=[
            pl.BlockSpec((nb, 1), lambda i: (i, 0)),
            pl.BlockSpec((1, NG), lambda i: (0, 0)),
        ],
        out_shape=[
            jax.ShapeDtypeStruct((N, 1), jnp.float32),
            jax.ShapeDtypeStruct((1, NG), jnp.float32),
        ],
    )(apart, wbig, mavg, prod, hold, rw1, rw2, e0, e1, batch2d)


def _block_mix(mix):
    lmap = [0, 1, 1, 1, 2, 2, 2, 2, 2]
    blocks = [[mix[lmap[m]] if m == m2 else jnp.zeros((C, C), jnp.float32)
               for m2 in range(SH_DIM)] for m in range(SH_DIM)]
    return jnp.block(blocks)


def _avg_mat():
    import numpy as np
    m = np.zeros((AROW, 3 * C), np.float32)
    lmap = [0, 1, 1, 1, 2, 2, 2, 2, 2]
    width = [1.0, 3.0, 5.0]
    for sh_m in range(SH_DIM):
        l = lmap[sh_m]
        for c in range(C):
            m[sh_m * C + c, l * C + c] = 1.0 / width[l]
    return jnp.asarray(m)


def kernel(positions, node_attrs, shifts, W_embed, atomic_energies,
           rW1_0, rW2_0, rW3_0, mix_0, prod_0, read_0,
           rW1_1, rW2_1, rW3_1, mix_1, prod_1, readf_W1, readf_W2,
           edge_index, batch):
    sender = edge_index[0]
    receiver = edge_index[1]
    pos4 = jnp.pad(positions, ((0, 0), (0, 1)))
    shifts_t = shifts.T
    zeros_a = jnp.zeros((N, AROW), jnp.float32)
    wbig0 = _block_mix(mix_0)
    wbig1 = _block_mix(mix_1)
    mavg = _avg_mat()
    ae_col = atomic_energies[:, None]
    batch2d = batch[:, None]

    w1c = jnp.concatenate([rW1_0, rW1_1], axis=1)
    z64 = jnp.zeros((64, 64), jnp.float32)
    w2c = jnp.block([[rW2_0, z64], [z64, rW2_1]])
    z48 = jnp.zeros((64, 48), jnp.float32)
    w3c = jnp.block([[rW3_0, z48], [z48, rW3_1]])

    ea = 163840
    eb_n = E - ea
    ei3 = edge_index.reshape(2, E // 80, 80)
    h0, e0 = _embed_call(node_attrs, W_embed, ae_col)
    dvec_a = _geom_call(pos4.reshape(-1), edge_index, 0, ea)
    ed_a = _edge_call(dvec_a, shifts_t, w1c, w2c, w3c, 0, ea)
    dvec_b = _geom_call(pos4.reshape(-1), edge_index, ea, eb_n)
    ap0a = _msg_call(0, 0, ea // NW, 80, ed_a, h0, ei3, zeros_a)
    ed_b = _edge_call(dvec_b, shifts_t, w1c, w2c, w3c, ea, eb_n)
    ap0 = _msg_call(0, ea, eb_n // NW, 80, ed_b, h0, ei3, ap0a,
                    chained=True)
    h1, e1 = _node_call(ap0, wbig0, mavg, prod_0, h0, read_0, read_0)
    ap1a = _msg_call(64, 0, ea // NW, 80, ed_a, h1, ei3, zeros_a)
    ap1 = _msg_call(64, ea, eb_n // NW, 80, ed_b, h1, ei3, ap1a,
                    chained=True)
    ne2d, tot2d = _node2_call(ap1, wbig1, mavg, prod_1, h1,
                              readf_W1, readf_W2, e0, e1, batch2d)
    return tot2d[0], ne2d[:, 0]


# speedup vs baseline: 1.0018x; 1.0018x over previous
import functools

import jax
import jax.numpy as jnp
from jax import lax
from jax.experimental import pallas as pl
from jax.experimental.pallas import tpu as pltpu
from jax.experimental.pallas import tpu_sc as plsc

N = 10000
E = 320000
NUM_ELEM = 10
C = 16
NG = 16
R_MAX = 5.0
NUM_BESSEL = 8
AVG_NEIGH = 32.0
SH_DIM = 9
AROW = SH_DIM * C

NC = 2
NS = 16
NW = NC * NS
EPW = E // NW
NPT = N // NS

@functools.cache
def _sc_mesh():
    return plsc.VectorSubcoreMesh(core_axis_name="c", subcore_axis_name="s",
                                  num_cores=NC, num_subcores=NS)


_SC_PARAMS = pltpu.CompilerParams(needs_layout_passes=False,
                                  use_tc_tiling_on_sc=False)


def _silu(x):
    return x / (1.0 + jnp.exp(-x))


def _geom_body(estart, ne, pos_hbm, ei_hbm, dvec_hbm, pos_v, sidx_v,
               ridx_v, dbuf_v):
    epw = ne // NW
    wid = lax.axis_index("s") * NC + lax.axis_index("c")
    base = wid * epw
    gbase = estart + base
    pltpu.sync_copy(pos_hbm, pos_v)
    pltpu.sync_copy(ei_hbm.at[0, pl.ds(gbase, epw)], sidx_v)
    pltpu.sync_copy(ei_hbm.at[1, pl.ds(gbase, epw)], ridx_v)

    def grp(g, _):
        off = g * 16
        si = sidx_v[pl.ds(off, 16)] * 4
        ri = ridx_v[pl.ds(off, 16)] * 4
        for c in range(3):
            ps = plsc.load_gather(pos_v, [si + c])
            pr = plsc.load_gather(pos_v, [ri + c])
            dbuf_v[c, pl.ds(off, 16)] = pr - ps
        return 0

    lax.fori_loop(0, epw // 16, grp, 0)
    pltpu.sync_copy(dbuf_v, dvec_hbm.at[:, pl.ds(base, epw)])


def _geom_call(pos4flat, edge_index, estart, ne):
    epw = ne // NW
    return pl.kernel(
        functools.partial(_geom_body, estart, ne),
        out_type=jax.ShapeDtypeStruct((3, ne), jnp.float32),
        mesh=_sc_mesh(),
        compiler_params=_SC_PARAMS,
        scratch_types=[
            pltpu.VMEM((N * 4,), jnp.float32),
            pltpu.VMEM((epw,), jnp.int32),
            pltpu.VMEM((epw,), jnp.int32),
            pltpu.VMEM((3, epw), jnp.float32),
        ],
    )(pos4flat, edge_index)


def _edge_body(dvec_ref, shifts_ref, w10, w20, w30, out_ref):
    x = dvec_ref[0:1, :] + shifts_ref[0:1, :]
    y = dvec_ref[1:2, :] + shifts_ref[1:2, :]
    z = dvec_ref[2:3, :] + shifts_ref[2:3, :]
    r2 = x * x + y * y + z * z + 1e-12
    r = jnp.sqrt(r2)
    inv_r = 1.0 / r
    ux = x * inv_r
    uy = y * inv_r
    uz = z * inv_r
    s3 = 3.0 ** 0.5
    s5 = 5.0 ** 0.5
    s15 = 15.0 ** 0.5
    sh = jnp.concatenate([
        s3 * ux, s3 * uy, s3 * uz,
        s15 * ux * uy, s15 * uy * uz, 0.5 * s5 * (3.0 * uz * uz - 1.0),
        s15 * ux * uz, 0.5 * s15 * (ux * ux - uy * uy)], axis=0)
    n = (lax.broadcasted_iota(jnp.int32, (NUM_BESSEL, 1), 0) + 1
         ).astype(jnp.float32)
    pref = (2.0 / R_MAX) ** 0.5
    rb = pref * jnp.sin(n * (jnp.pi / R_MAX) * r) * inv_r
    xc = r * (1.0 / R_MAX)
    x2 = xc * xc
    x4 = x2 * x2
    x5 = x4 * xc
    x6 = x5 * xc
    x7 = x6 * xc
    env = 1.0 - 21.0 * x5 + 35.0 * x6 - 15.0 * x7
    env = jnp.where(xc < 1.0, env, 0.0)
    rb = rb * env
    g = jnp.concatenate([rb, sh], axis=0)
    gt = jnp.transpose(g, (1, 0))
    rb = gt[:, 0:NUM_BESSEL]
    sh = gt[:, NUM_BESSEL:16]
    rr = _silu(jnp.dot(rb, w10[...], preferred_element_type=jnp.float32))
    rr = _silu(jnp.dot(rr, w20[...], preferred_element_type=jnp.float32))
    rr = jnp.dot(rr, w30[...], preferred_element_type=jnp.float32)
    out_ref[:, 0:48] = rr[:, 0:48]
    out_ref[:, 48:56] = sh
    out_ref[:, 56:64] = jnp.zeros_like(out_ref[:, 56:64])
    out_ref[:, 64:112] = rr[:, 48:96]
    out_ref[:, 112:120] = sh
    out_ref[:, 120:128] = jnp.zeros_like(out_ref[:, 120:128])


def _wspec(a):
    return pl.BlockSpec(a.shape, lambda i: (0,) * a.ndim)


def _edge_call(dvec, shifts4, w10, w20, w30, lo, ne):
    eb = 1280
    ofs = lo // eb
    return pl.pallas_call(
        _edge_body,
        grid=ne // eb,
        in_specs=[
            pl.BlockSpec((3, eb), lambda i: (0, i)),
            pl.BlockSpec((3, eb), lambda i: (0, i + ofs)),
            _wspec(w10), _wspec(w20), _wspec(w30),
        ],
        out_specs=pl.BlockSpec((eb, 128), lambda i: (i, 0)),
        out_shape=jax.ShapeDtypeStruct((ne, 128), jnp.float32),
    )(dvec, shifts4, w10, w20, w30)


CHUNK = 80


def _msg_body(roff, estart, epw, chunk, chained, edata_hbm, htab_hbm,
              ei3_hbm, init_hbm, apart_hbm, ed0, ed1, hs0, hs1, eib0,
              eib1, msg0, msg1, acc_sh, se0, se1, si0, si1, sg0, sg1,
              sc0, sc1):
    nch = epw // chunk
    cid = lax.axis_index("c")
    sid = lax.axis_index("s")
    wid = sid * NC + cid
    base = wid * epw
    gc0 = (estart + base) // chunk
    ed = (ed0, ed1)
    hs = (hs0, hs1)
    eib = (eib0, eib1)
    msg = (msg0, msg1)
    se = (se0, se1)
    si = (si0, si1)
    sg = (sg0, sg1)
    sc = (sc0, sc1)

    for t in range(NS):
        lo = 624 * t
        sz = 624 if t < NS - 1 else N - 624 * (NS - 1)

        @pl.when(sid == t)
        def _(lo=lo, sz=sz):
            if chained:
                pltpu.sync_copy(init_hbm.at[cid, pl.ds(lo, sz)],
                                acc_sh.at[pl.ds(lo, sz)])
            else:
                pltpu.sync_copy(init_hbm.at[pl.ds(lo, sz)],
                                acc_sh.at[pl.ds(lo, sz)])

    plsc.subcore_barrier()

    def lin_issue(j, b):
        cb = base + j * chunk
        pltpu.async_copy(edata_hbm.at[pl.ds(cb, chunk), pl.ds(roff, 64)],
                         ed[b], se[b])
        pltpu.async_copy(ei3_hbm.at[:, gc0 + j, :], eib[b], si[b])

    def wait_si(b):
        pltpu.make_async_copy(ei3_hbm.at[:, gc0, :], eib[b],
                              si[b]).wait()

    def gather_issue(b):
        pltpu.async_copy(htab_hbm.at[eib[b].at[0]], hs[b], sg[b])

    def process(b):
        pltpu.make_async_copy(
            edata_hbm.at[pl.ds(base, chunk), pl.ds(roff, 64)], ed[b],
            se[b]).wait()
        pltpu.make_async_copy(htab_hbm.at[eib[b].at[0]], hs[b],
                              sg[b]).wait()
        ed_v = ed[b]
        hs_v = hs[b]
        msg_v = msg[b]

        def edge(i, _):
            hv = hs_v[i, :]
            t0 = ed_v[i, pl.ds(0, 16)] * hv
            t1 = ed_v[i, pl.ds(16, 16)] * hv
            t2 = ed_v[i, pl.ds(32, 16)] * hv
            shv = ed_v[i, pl.ds(48, 16)]
            msg_v[i, pl.ds(0, 16)] = t0
            msg_v[i, pl.ds(16, 16)] = t1 * shv[0]
            msg_v[i, pl.ds(32, 16)] = t1 * shv[1]
            msg_v[i, pl.ds(48, 16)] = t1 * shv[2]
            msg_v[i, pl.ds(64, 16)] = t2 * shv[3]
            msg_v[i, pl.ds(80, 16)] = t2 * shv[4]
            msg_v[i, pl.ds(96, 16)] = t2 * shv[5]
            msg_v[i, pl.ds(112, 16)] = t2 * shv[6]
            msg_v[i, pl.ds(128, 16)] = t2 * shv[7]
            return 0

        lax.fori_loop(0, chunk, edge, 0, unroll=2)
        pltpu.async_copy(msg_v, acc_sh.at[eib[b].at[1]], sc[b], add=True)

    def wait_sc(b):
        pltpu.make_async_copy(msg[b], acc_sh.at[eib[b].at[1]],
                              sc[b]).wait()

    lin_issue(0, 0)
    wait_si(0)
    gather_issue(0)
    lin_issue(1, 1)

    def pair(i, _):
        j = 2 * i
        wait_si(1)
        gather_issue(1)
        process(0)
        process(1)

        @pl.when(j + 2 < nch)
        def _():
            wait_sc(0)
            lin_issue(j + 2, 0)
            wait_si(0)
            gather_issue(0)

        @pl.when(j + 3 < nch)
        def _():
            wait_sc(1)
            lin_issue(j + 3, 1)

        return 0

    lax.fori_loop(0, nch // 2, pair, 0)
    if nch % 2 == 1:
        process(0)
    wait_sc(0)
    wait_sc(1)
    plsc.subcore_barrier()
    for t in range(NS):
        lo = 624 * t
        sz = 624 if t < NS - 1 else N - 624 * (NS - 1)

        @pl.when(sid == t)
        def _(lo=lo, sz=sz):
            pltpu.sync_copy(acc_sh.at[pl.ds(lo, sz)],
                            apart_hbm.at[cid, pl.ds(lo, sz)])


def _msg_call(roff, estart, epw, chunk, edata, htab, edge_index, init,
              chained=False):
    body = functools.partial(_msg_body, roff, estart, epw, chunk, chained)
    return pl.kernel(
        body,
        out_type=jax.ShapeDtypeStruct((NC, N, AROW), jnp.float32),
        mesh=_sc_mesh(),
        compiler_params=_SC_PARAMS,
        scratch_types=[
            pltpu.VMEM((chunk, 64), jnp.float32),
            pltpu.VMEM((chunk, 64), jnp.float32),
            pltpu.VMEM((chunk, 16), jnp.float32),
            pltpu.VMEM((chunk, 16), jnp.float32),
            pltpu.VMEM((2, chunk), jnp.int32),
            pltpu.VMEM((2, chunk), jnp.int32),
            pltpu.VMEM((chunk, AROW), jnp.float32),
            pltpu.VMEM((chunk, AROW), jnp.float32),
            pltpu.VMEM_SHARED((N, AROW), jnp.float32),
        ] + [pltpu.SemaphoreType.DMA] * 8,
    )(edata, htab, edge_index, init)


def _embed_body(na_ref, we_ref, ae_ref, h0_ref, e0_ref):
    na = na_ref[...]
    h0_ref[...] = jnp.dot(na, we_ref[...], preferred_element_type=jnp.float32)
    e0_ref[...] = jnp.dot(na, ae_ref[...], preferred_element_type=jnp.float32)


def _embed_call(node_attrs, w_embed, ae_col):
    nb = 1000
    return pl.pallas_call(
        _embed_body,
        grid=N // nb,
        in_specs=[
            pl.BlockSpec((nb, NUM_ELEM), lambda i: (i, 0)),
            _wspec(w_embed),
            _wspec(ae_col),
        ],
        out_specs=[
            pl.BlockSpec((nb, C), lambda i: (i, 0)),
            pl.BlockSpec((nb, 1), lambda i: (i, 0)),
        ],
        out_shape=[
            jax.ShapeDtypeStruct((N, C), jnp.float32),
            jax.ShapeDtypeStruct((N, 1), jnp.float32),
        ],
    )(node_attrs, w_embed, ae_col)


def _node_body(apart_ref, wbig_ref, mavg_ref, prod_ref,
               hold_ref, rw1_ref, rw2_ref, hnew_ref, e_ref):
    a = (apart_ref[0] + apart_ref[1]) * (1.0 / AVG_NEIGH)
    amix = jnp.dot(a, wbig_ref[...], preferred_element_type=jnp.float32)
    inv = jnp.dot(amix * amix, mavg_ref[...],
                  preferred_element_type=jnp.float32)
    s = (amix[:, 0:16] + jnp.dot(inv, prod_ref[...],
                                 preferred_element_type=jnp.float32)
         + hold_ref[...])
    hnew_ref[...] = s
    e_ref[...] = jnp.dot(s, rw1_ref[...], preferred_element_type=jnp.float32)


def _node_call(apart, wbig, mavg, prod, hold, rw1, rw2):
    nb = 1000
    return pl.pallas_call(
        _node_body,
        grid=N // nb,
        in_specs=[
            pl.BlockSpec((NC, nb, AROW), lambda i: (0, i, 0)),
            _wspec(wbig), _wspec(mavg), _wspec(prod),
            pl.BlockSpec((nb, C), lambda i: (i, 0)),
            _wspec(rw1), _wspec(rw2),
        ],
        out_specs=[
            pl.BlockSpec((nb, C), lambda i: (i, 0)),
            pl.BlockSpec((nb, 1), lambda i: (i, 0)),
        ],
        out_shape=[
            jax.ShapeDtypeStruct((N, C), jnp.float32),
            jax.ShapeDtypeStruct((N, 1), jnp.float32),
        ],
    )(apart, wbig, mavg, prod, hold, rw1, rw2)


def _node2_body(apart_ref, wbig_ref, mavg_ref, prod_ref,
                hold_ref, rw1_ref, rw2_ref, e0_ref, e1_ref, batch_ref,
                ne_ref, tot_ref):
    a = (apart_ref[0] + apart_ref[1]) * (1.0 / AVG_NEIGH)
    amix = jnp.dot(a, wbig_ref[...], preferred_element_type=jnp.float32)
    inv = jnp.dot(amix * amix, mavg_ref[...],
                  preferred_element_type=jnp.float32)
    s = (amix[:, 0:16] + jnp.dot(inv, prod_ref[...],
                                 preferred_element_type=jnp.float32)
         + hold_ref[...])
    t = _silu(jnp.dot(s, rw1_ref[...], preferred_element_type=jnp.float32))
    e2 = jnp.dot(t, rw2_ref[...], preferred_element_type=jnp.float32)
    ne = e0_ref[...] + e1_ref[...] + e2
    ne_ref[...] = ne
    gi = lax.broadcasted_iota(jnp.int32, (1, NG), 1)
    oh = (batch_ref[...] == gi).astype(jnp.float32)
    part = jnp.sum(oh * ne, axis=0, keepdims=True)

    @pl.when(pl.program_id(0) == 0)
    def _():
        tot_ref[...] = jnp.zeros_like(tot_ref)

    tot_ref[...] += part


def _node2_call(apart, wbig, mavg, prod, hold, rw1, rw2, e0, e1,
                batch2d):
    nb = 1000
    return pl.pallas_call(
        _node2_body,
        grid=N // nb,
        in_specs=[
            pl.BlockSpec((NC, nb, AROW), lambda i: (0, i, 0)),
            _wspec(wbig), _wspec(mavg), _wspec(prod),
            pl.BlockSpec((nb, C), lambda i: (i, 0)),
            _wspec(rw1), _wspec(rw2),
            pl.BlockSpec((nb, 1), lambda i: (i, 0)),
            pl.BlockSpec((nb, 1), lambda i: (i, 0)),
            pl.BlockSpec((nb, 1), lambda i: (i, 0)),
        ],
        out_specs=[
            pl.BlockSpec((nb, 1), lambda i: (i, 0)),
            pl.BlockSpec((1, NG), lambda i: (0, 0)),
        ],
        out_shape=[
            jax.ShapeDtypeStruct((N, 1), jnp.float32),
            jax.ShapeDtypeStruct((1, NG), jnp.float32),
        ],
    )(apart, wbig, mavg, prod, hold, rw1, rw2, e0, e1, batch2d)


def _block_mix(mix):
    lmap = [0, 1, 1, 1, 2, 2, 2, 2, 2]
    blocks = [[mix[lmap[m]] if m == m2 else jnp.zeros((C, C), jnp.float32)
               for m2 in range(SH_DIM)] for m in range(SH_DIM)]
    return jnp.block(blocks)


def _avg_mat():
    import numpy as np
    m = np.zeros((AROW, 3 * C), np.float32)
    lmap = [0, 1, 1, 1, 2, 2, 2, 2, 2]
    width = [1.0, 3.0, 5.0]
    for sh_m in range(SH_DIM):
        l = lmap[sh_m]
        for c in range(C):
            m[sh_m * C + c, l * C + c] = 1.0 / width[l]
    return jnp.asarray(m)


def kernel(positions, node_attrs, shifts, W_embed, atomic_energies,
           rW1_0, rW2_0, rW3_0, mix_0, prod_0, read_0,
           rW1_1, rW2_1, rW3_1, mix_1, prod_1, readf_W1, readf_W2,
           edge_index, batch):
    pos4 = jnp.pad(positions, ((0, 0), (0, 1)))
    shifts_t = shifts.T
    zeros_a = jnp.zeros((N, AROW), jnp.float32)
    wbig0 = _block_mix(mix_0)
    wbig1 = _block_mix(mix_1)
    mavg = _avg_mat()
    ae_col = atomic_energies[:, None]
    batch2d = batch[:, None]

    w1c = jnp.concatenate([rW1_0, rW1_1], axis=1)
    z64 = jnp.zeros((64, 64), jnp.float32)
    w2c = jnp.block([[rW2_0, z64], [z64, rW2_1]])
    z48 = jnp.zeros((64, 48), jnp.float32)
    w3c = jnp.block([[rW3_0, z48], [z48, rW3_1]])

    ea = 163840
    eb_n = E - ea
    ei3 = edge_index.reshape(2, E // 80, 80)
    h0, e0 = _embed_call(node_attrs, W_embed, ae_col)
    dvec_a = _geom_call(pos4.reshape(-1), edge_index, 0, ea)
    ed_a = _edge_call(dvec_a, shifts_t, w1c, w2c, w3c, 0, ea)
    dvec_b = _geom_call(pos4.reshape(-1), edge_index, ea, eb_n)
    ap0a = _msg_call(0, 0, ea // NW, 80, ed_a, h0, ei3, zeros_a)
    ed_b = _edge_call(dvec_b, shifts_t, w1c, w2c, w3c, ea, eb_n)
    ap0 = _msg_call(0, ea, eb_n // NW, 80, ed_b, h0, ei3, ap0a,
                    chained=True)
    h1, e1 = _node_call(ap0, wbig0, mavg, prod_0, h0, read_0, read_0)
    ap1a = _msg_call(64, 0, ea // NW, 80, ed_a, h1, ei3, zeros_a)
    ap1 = _msg_call(64, ea, eb_n // NW, 80, ed_b, h1, ei3, ap1a,
                    chained=True)
    ne2d, tot2d = _node2_call(ap1, wbig1, mavg, prod_1, h1,
                              readf_W1, readf_W2, e0, e1, batch2d)
    return tot2d[0], ne2d[:, 0]
